# baseline (speedup 1.0000x reference)
import jax
import jax.numpy as jnp
import numpy as np
from jax.experimental import pallas as pl

_C = 1000
_F = 8
_N = 1024


def _coalition_masks():
    def gen(key):
        importance = jnp.exp(-0.1 * jnp.arange(_F, dtype=jnp.float32))
        probs = jax.nn.softmax(importance)
        k1, k2 = jax.random.split(key)
        sizes = jax.random.randint(k1, (_C,), 1, _F)
        gumbel = jax.random.gumbel(k2, (_C, _F))
        scores = jnp.log(probs)[None, :] + gumbel
        order = jnp.argsort(-scores, axis=1)
        ranks = jnp.argsort(order, axis=1)
        return (ranks < sizes[:, None]).astype(jnp.float32)

    k_spec, k_spat, k_temp = jax.random.split(jax.random.key(42), 3)
    return jnp.stack([gen(k) for k in (k_spec, k_spat, k_temp)])


def _explainer_kernel(x0_ref, x1_ref, lam_ref, u_ref, masks_ref, out_ref):
    hi = jax.lax.Precision.HIGHEST

    def dot_t(a, b):
        return jax.lax.dot_general(a, b, (((0,), (0,)), ((), ())),
                                   precision=hi,
                                   preferred_element_type=jnp.float32)

    def dot(a, b):
        return jax.lax.dot_general(a, b, (((1,), (0,)), ((), ())),
                                   precision=hi,
                                   preferred_element_type=jnp.float32)

    e_lat, e_tmp = [], []
    for b in range(2):
        u_b = u_ref[b]
        gain = jnp.exp(-lam_ref[b])[:, None]
        for acc, y in ((e_lat, x1_ref[b]),
                       (e_tmp, x1_ref[b] - x0_ref[b])):
            proj = dot_t(u_b, y)
            h = dot(u_b, gain * proj)
            acc.append(jnp.sum(h * h, axis=0, keepdims=True) / (_N * _F))
    e1 = jnp.concatenate(e_lat, axis=0)
    et = jnp.concatenate(e_tmp, axis=0)

    ones_c = jnp.ones((_C, 1), jnp.float32)
    outs = []
    for v, e_v in ((0, e1), (1, e1), (2, et)):
        m = masks_ref[v]
        gram = dot_t(m, m)
        cw_row = dot_t(ones_c, m)
        cw_col = dot_t(m, ones_c)
        cwo_row = _C - cw_row
        w = (gram / jnp.maximum(cw_row, 1.0)
             - (cw_col - gram) / jnp.maximum(cwo_row, 1.0))
        valid = ((cw_row > 0.0) & (cwo_row > 0.0)).astype(jnp.float32)
        outs.append(dot(e_v, w * valid))
    out_ref[...] = jnp.concatenate(outs, axis=1)


def kernel(x0, x1, eigenvalues, eigenvectors):
    masks = _coalition_masks()
    return pl.pallas_call(
        _explainer_kernel,
        out_shape=jax.ShapeDtypeStruct((x0.shape[0], 24), jnp.float32),
    )(x0, x1, eigenvalues, eigenvectors, masks)


# speedup vs baseline: 2.8260x; 2.8260x over previous
import functools

import jax
import jax.numpy as jnp
import numpy as np
from jax import lax
from jax.experimental import pallas as pl
from jax.experimental.pallas import tpu as pltpu
from jax.experimental.pallas import tpu_sc as plsc

_C = 1000
_F = 8
_N = 1024
_CP = 1024
_NC = 2
_NS = 16
_L = 16


def _coalition_masks() -> np.ndarray:
    def gen(key):
        importance = jnp.exp(-0.1 * jnp.arange(_F, dtype=jnp.float32))
        probs = jax.nn.softmax(importance)
        k1, k2 = jax.random.split(key)
        sizes = jax.random.randint(k1, (_C,), 1, _F)
        gumbel = jax.random.gumbel(k2, (_C, _F))
        scores = jnp.log(probs)[None, :] + gumbel
        order = jnp.argsort(-scores, axis=1)
        ranks = jnp.argsort(order, axis=1)
        return (ranks < sizes[:, None]).astype(jnp.float32)

    k_spec, k_spat, k_temp = jax.random.split(jax.random.key(42), 3)
    return np.stack([np.asarray(gen(k)) for k in (k_spec, k_spat, k_temp)])


_MASKS = _coalition_masks()

_MASKS_T = np.zeros((3 * _F, _CP), np.float32)
_MASKS_T[:, :_C] = _MASKS.transpose(0, 2, 1).reshape(3 * _F, _C)


def _sc_stats_body(masks_hbm, out_hbm, rows_v, mf_v, res_v):
    wid = lax.axis_index("s") * _NC + lax.axis_index("c")

    @pl.when(wid < 3 * _F)
    def _():
        v = wid // _F
        pltpu.sync_copy(masks_hbm.at[pl.ds(v * _F, _F)], rows_v)
        pltpu.sync_copy(masks_hbm.at[pl.ds(wid, 1)], mf_v)

        def body(ci, accs):
            sl = pl.ds(ci * _L, _L)
            mf = mf_v[0, sl]
            new = tuple(accs[i] + mf * rows_v[i, sl] for i in range(_F))
            return new + (accs[_F] + mf,)

        zero = jnp.zeros((_L,), jnp.float32)
        accs = lax.fori_loop(0, _CP // _L, body, (zero,) * (_F + 1))

        for i in range(_F + 1):
            res_v[i, :] = accs[i]
        pltpu.sync_copy(res_v, out_hbm.at[wid])


_sc_mask_stats = functools.partial(
    pl.kernel,
    out_type=jax.ShapeDtypeStruct((3 * _F, _F + 1, _L), jnp.float32),
    scratch_types=[
        pltpu.VMEM((_F, _CP), jnp.float32),
        pltpu.VMEM((1, _CP), jnp.float32),
        pltpu.VMEM((_F + 1, _L), jnp.float32),
    ],
)


def _tc_explainer_kernel(x0_ref, x1_ref, lam_ref, u_ref, stats_ref, out_ref):
    hi = jax.lax.Precision.HIGHEST

    def dot_t(a, b):
        return jax.lax.dot_general(a, b, (((0,), (0,)), ((), ())),
                                   precision=hi,
                                   preferred_element_type=jnp.float32)

    def dot(a, b):
        return jax.lax.dot_general(a, b, (((1,), (0,)), ((), ())),
                                   precision=hi,
                                   preferred_element_type=jnp.float32)

    e_lat, e_tmp = [], []
    for b in range(2):
        u_b = u_ref[b]
        gain = jnp.exp(-lam_ref[b])[:, None]
        for acc, y in ((e_lat, x1_ref[b]),
                       (e_tmp, x1_ref[b] - x0_ref[b])):
            proj = dot_t(u_b, y)
            h = dot(u_b, gain * proj)
            acc.append(jnp.sum(h * h, axis=0, keepdims=True) / (_N * _F))
    e1 = jnp.concatenate(e_lat, axis=0)
    et = jnp.concatenate(e_tmp, axis=0)

    stats = jnp.sum(stats_ref[...], axis=2)
    eye8 = jnp.eye(_F, dtype=jnp.float32)
    outs = []
    for v, e_v in ((0, e1), (1, e1), (2, et)):
        block = stats[v * _F:(v + 1) * _F, :]
        gram = block[:, 0:_F]
        cw_col = block[:, _F:_F + 1]
        cw_row = dot_t(cw_col, eye8)
        cwo_row = _C - cw_row
        w = (gram / jnp.maximum(cw_row, 1.0)
             - (cw_col - gram) / jnp.maximum(cwo_row, 1.0))
        valid = ((cw_row > 0.0) & (cwo_row > 0.0)).astype(jnp.float32)
        outs.append(dot(e_v, w * valid))
    out_ref[...] = jnp.concatenate(outs, axis=1)


def kernel(x0, x1, eigenvalues, eigenvectors):
    mesh = plsc.VectorSubcoreMesh(core_axis_name="c", subcore_axis_name="s",
                                  num_cores=_NC, num_subcores=_NS)
    stats = _sc_mask_stats(_sc_stats_body, mesh=mesh)(jnp.asarray(_MASKS_T))
    return pl.pallas_call(
        _tc_explainer_kernel,
        out_shape=jax.ShapeDtypeStruct((x0.shape[0], 24), jnp.float32),
    )(x0, x1, eigenvalues, eigenvectors, stats)
